# baseline (speedup 1.0000x reference)
import functools

import jax
import jax.numpy as jnp
from jax import lax
from jax.experimental import pallas as pl
from jax.experimental.pallas import tpu as pltpu
from jax.experimental.pallas import tpu_sc as plsc

NTILES = 32
L = 16
_F32 = jnp.float32


def _mesh():
    return plsc.VectorSubcoreMesh(
        core_axis_name="c", subcore_axis_name="s", num_cores=2, num_subcores=16
    )


_SC_PARAMS = pltpu.CompilerParams(needs_layout_passes=False)


def _wid():
    return lax.axis_index("s") * 2 + lax.axis_index("c")


def _zero_1d(ref, n, unroll=5):
    z = jnp.zeros((L,), _F32)

    @plsc.parallel_loop(0, n // L, unroll=unroll)
    def _(i):
        ref[pl.ds(i * L, L)] = z


def _make_deg(E, N):
    ept = E // NTILES

    @functools.partial(
        pl.kernel,
        out_type=jax.ShapeDtypeStruct((NTILES, N), _F32),
        mesh=_mesh(),
        compiler_params=_SC_PARAMS,
        scratch_types=[
            pltpu.VMEM((ept,), jnp.int32),
            pltpu.VMEM((ept,), _F32),
            pltpu.VMEM((N,), _F32),
        ],
    )
    def deg_k(dst_hbm, ew_hbm, out_hbm, dstv, ewv, acc):
        wid = _wid()
        base = wid * ept
        pltpu.sync_copy(dst_hbm.at[pl.ds(base, ept)], dstv)
        pltpu.sync_copy(ew_hbm.at[pl.ds(base, ept)], ewv)
        _zero_1d(acc, N)

        @plsc.parallel_loop(0, ept // L, unroll=5)
        def _(i):
            d = dstv[pl.ds(i * L, L)]
            w = ewv[pl.ds(i * L, L)]
            plsc.addupdate_scatter(acc, [d], w)
        pltpu.sync_copy(acc, out_hbm.at[wid])

    return deg_k


def _make_agg(E, N, F, ft=4, ce=4000, unroll=10):
    split = (NTILES * ft) // F
    es = E // split
    nch = es // ce
    npairs = nch // 2
    assert nch % 2 == 0 and (ce // L) % unroll == 0
    scratch = (
        [pltpu.VMEM((N,), _F32) for _ in range(2 * ft)]
        + 2 * [pltpu.VMEM((ce,), jnp.int32), pltpu.VMEM((ce,), _F32)]
        + [pltpu.SemaphoreType.DMA, pltpu.SemaphoreType.DMA]
    )

    @functools.partial(
        pl.kernel,
        out_type=jax.ShapeDtypeStruct((split, F, N), _F32),
        mesh=_mesh(),
        compiler_params=_SC_PARAMS,
        scratch_types=scratch,
    )
    def agg_k(sd_hbm, ew_hbm, y_hbm, out_hbm, *scr):
        cols = scr[:ft]
        accs = scr[ft:2 * ft]
        bufs = (scr[2 * ft:2 * ft + 2], scr[2 * ft + 2:2 * ft + 4])
        sems = scr[2 * ft + 4:2 * ft + 6]
        wid = _wid()
        sub = wid % split
        f0 = (wid // split) * ft
        e0 = sub * es

        def start(c, b):
            sdb, ewb = bufs[b]
            pltpu.async_copy(sd_hbm.at[pl.ds(e0 + c * ce, ce)], sdb, sems[b])
            pltpu.async_copy(ew_hbm.at[pl.ds(e0 + c * ce, ce)], ewb, sems[b])

        def wait(c, b):
            sdb, ewb = bufs[b]
            pltpu.make_async_copy(sd_hbm.at[pl.ds(e0 + c * ce, ce)], sdb, sems[b]).wait()
            pltpu.make_async_copy(ew_hbm.at[pl.ds(e0 + c * ce, ce)], ewb, sems[b]).wait()

        def compute(b):
            sdb, ewb = bufs[b]

            @plsc.parallel_loop(0, ce // L, unroll=unroll)
            def _(i):
                base = i * L
                p = sdb[pl.ds(base, L)]
                w = ewb[pl.ds(base, L)]
                s = jnp.bitwise_and(p, 0xFFFF)
                d = jnp.right_shift(p, 16)
                for j in range(ft):
                    v = plsc.load_gather(cols[j], [s])
                    plsc.addupdate_scatter(accs[j], [d], v * w)

        start(0, 0)
        for j in range(ft):
            pltpu.sync_copy(y_hbm.at[f0 + j], cols[j])
            _zero_1d(accs[j], N)

        def pair(c, carry):
            start(2 * c + 1, 1)
            wait(2 * c, 0)
            compute(0)

            @pl.when(c < npairs - 1)
            def _():
                start(2 * c + 2, 0)

            wait(2 * c + 1, 1)
            compute(1)
            return carry

        lax.fori_loop(0, npairs, pair, 0)
        for j in range(ft):
            pltpu.sync_copy(accs[j], out_hbm.at[sub, f0 + j])

    return agg_k


def _tc_pack(edge_index):
    E = edge_index.shape[1]

    def body(er, p_ref):
        p_ref[...] = jnp.bitwise_or(
            er[0:1, :], jnp.left_shift(er[1:2, :], 16)
        )

    return pl.pallas_call(
        body, out_shape=jax.ShapeDtypeStruct((1, E), jnp.int32)
    )(edge_index)


def _tc_first(degp, xT, W1T):
    F, N = W1T.shape[0], xT.shape[1]

    def body(dp, xr, wr, y_ref, dinv_ref):
        deg = jnp.sum(dp[...], axis=0, keepdims=True) + 1.0
        dinv = lax.rsqrt(jnp.maximum(deg, 1e-12))
        y = jnp.dot(wr[...], xr[...], preferred_element_type=_F32) * dinv
        y_ref[...] = y
        dinv_ref[...] = dinv

    return pl.pallas_call(
        body,
        out_shape=(
            jax.ShapeDtypeStruct((F, N), _F32),
            jax.ShapeDtypeStruct((1, N), _F32),
        ),
    )(degp, xT, W1T)


def _tc_mid(agg, yT, dinv, WT, bcol):
    F, N = WT.shape[0], yT.shape[1]

    def body(ar, yr, dr, wr, br, o_ref):
        agg = jnp.sum(ar[...], axis=0)
        h = jnp.maximum(dr[...] * (agg + yr[...]) + br[...], 0.0)
        o_ref[...] = jnp.dot(wr[...], h, preferred_element_type=_F32) * dr[...]

    return pl.pallas_call(
        body, out_shape=jax.ShapeDtypeStruct((F, N), _F32)
    )(agg, yT, dinv, WT, bcol)


def _tc_last(agg, yT, dinv, bcol):
    F, N = yT.shape

    def body(ar, yr, dr, br, emb_ref, gm_ref):
        embT = dr[...] * (jnp.sum(ar[...], axis=0) + yr[...]) + br[...]
        emb_ref[...] = jnp.transpose(embT)
        gm_ref[...] = jnp.mean(embT, axis=1, keepdims=True)

    return pl.pallas_call(
        body,
        out_shape=(
            jax.ShapeDtypeStruct((N, F), _F32),
            jax.ShapeDtypeStruct((F, 1), _F32),
        ),
    )(agg, yT, dinv, bcol)


def kernel(x, edge_index, edge_attr, W1, b1, W2, b2, W3, b3):
    N, D = x.shape
    E = edge_index.shape[1]
    H = W1.shape[1]
    O = W3.shape[1]

    dst = edge_index[1]
    ew = edge_attr[:, 0]
    xT = x.T
    W1T, W2T, W3T = W1.T, W2.T, W3.T
    b1c, b2c, b3c = b1.reshape(H, 1), b2.reshape(H, 1), b3.reshape(O, 1)

    sd = _tc_pack(edge_index).reshape(E)
    degp = _make_deg(E, N)(dst, ew)
    y1T, dinv = _tc_first(degp, xT, W1T)

    agg_h = _make_agg(E, N, H)
    agg1 = agg_h(sd, ew, y1T)
    y2T = _tc_mid(agg1, y1T, dinv, W2T, b1c)
    agg2 = agg_h(sd, ew, y2T)
    y3T = _tc_mid(agg2, y2T, dinv, W3T, b2c)
    agg3 = _make_agg(E, N, O)(sd, ew, y3T)
    emb, gsum = _tc_last(agg3, y3T, dinv, b3c)

    return emb, gsum.reshape(1, O)


# speedup vs baseline: 1.0295x; 1.0295x over previous
import functools

import jax
import jax.numpy as jnp
from jax import lax
from jax.experimental import pallas as pl
from jax.experimental.pallas import tpu as pltpu
from jax.experimental.pallas import tpu_sc as plsc

NTILES = 32
L = 16
_F32 = jnp.float32


def _mesh():
    return plsc.VectorSubcoreMesh(
        core_axis_name="c", subcore_axis_name="s", num_cores=2, num_subcores=16
    )


_SC_PARAMS = pltpu.CompilerParams(needs_layout_passes=False)


def _wid():
    return lax.axis_index("s") * 2 + lax.axis_index("c")


def _zero_1d(ref, n, unroll=5):
    z = jnp.zeros((L,), _F32)

    @plsc.parallel_loop(0, n // L, unroll=unroll)
    def _(i):
        ref[pl.ds(i * L, L)] = z


def _make_deg(E, N):
    ept = E // NTILES

    @functools.partial(
        pl.kernel,
        out_type=jax.ShapeDtypeStruct((NTILES, N), _F32),
        mesh=_mesh(),
        compiler_params=_SC_PARAMS,
        scratch_types=[
            pltpu.VMEM((ept,), jnp.int32),
            pltpu.VMEM((ept,), _F32),
            pltpu.VMEM((N,), _F32),
        ],
    )
    def deg_k(dst_hbm, ew_hbm, out_hbm, dstv, ewv, acc):
        wid = _wid()
        base = wid * ept
        pltpu.sync_copy(dst_hbm.at[pl.ds(base, ept)], dstv)
        pltpu.sync_copy(ew_hbm.at[pl.ds(base, ept)], ewv)
        _zero_1d(acc, N)

        @plsc.parallel_loop(0, ept // L, unroll=5)
        def _(i):
            d = dstv[pl.ds(i * L, L)]
            w = ewv[pl.ds(i * L, L)]
            plsc.addupdate_scatter(acc, [d], w)
        pltpu.sync_copy(acc, out_hbm.at[wid])

    return deg_k


def _make_agg(E, N, F, ft=4, ce=4000, unroll=10):
    split = (NTILES * ft) // F
    es = E // split
    nch = es // ce
    npairs = nch // 2
    assert nch % 2 == 0 and (ce // L) % unroll == 0
    scratch = (
        [pltpu.VMEM((N,), _F32) for _ in range(2 * ft)]
        + 2 * [pltpu.VMEM((ce,), jnp.int32), pltpu.VMEM((ce,), _F32)]
        + [pltpu.SemaphoreType.DMA, pltpu.SemaphoreType.DMA]
    )

    @functools.partial(
        pl.kernel,
        out_type=jax.ShapeDtypeStruct((split, F, N), _F32),
        mesh=_mesh(),
        compiler_params=_SC_PARAMS,
        scratch_types=scratch,
    )
    def agg_k(sd_hbm, ew_hbm, y_hbm, out_hbm, *scr):
        cols = scr[:ft]
        accs = scr[ft:2 * ft]
        bufs = (scr[2 * ft:2 * ft + 2], scr[2 * ft + 2:2 * ft + 4])
        sems = scr[2 * ft + 4:2 * ft + 6]
        wid = _wid()
        sub = wid % split
        f0 = (wid // split) * ft
        e0 = sub * es

        def start(c, b):
            sdb, ewb = bufs[b]
            pltpu.async_copy(sd_hbm.at[pl.ds(e0 + c * ce, ce)], sdb, sems[b])
            pltpu.async_copy(ew_hbm.at[pl.ds(e0 + c * ce, ce)], ewb, sems[b])

        def wait(c, b):
            sdb, ewb = bufs[b]
            pltpu.make_async_copy(sd_hbm.at[pl.ds(e0 + c * ce, ce)], sdb, sems[b]).wait()
            pltpu.make_async_copy(ew_hbm.at[pl.ds(e0 + c * ce, ce)], ewb, sems[b]).wait()

        def compute(b):
            sdb, ewb = bufs[b]

            @plsc.parallel_loop(0, ce // L, unroll=unroll)
            def _(i):
                base = i * L
                p = sdb[pl.ds(base, L)]
                w = ewb[pl.ds(base, L)]
                s = jnp.bitwise_and(p, 0xFFFF)
                d = jnp.right_shift(p, 16)
                for j in range(ft):
                    v = plsc.load_gather(cols[j], [s])
                    plsc.addupdate_scatter(accs[j], [d], v * w)

        start(0, 0)
        for j in range(ft):
            pltpu.sync_copy(y_hbm.at[f0 + j], cols[j])
            _zero_1d(accs[j], N)

        def pair(c, carry):
            start(2 * c + 1, 1)
            wait(2 * c, 0)
            compute(0)

            @pl.when(c < npairs - 1)
            def _():
                start(2 * c + 2, 0)

            wait(2 * c + 1, 1)
            compute(1)
            return carry

        lax.fori_loop(0, npairs, pair, 0)
        for j in range(ft):
            pltpu.sync_copy(accs[j], out_hbm.at[sub, f0 + j])

    return agg_k


def _tc_pack(edge_index):
    E = edge_index.shape[1]

    def body(er, p_ref):
        p_ref[...] = jnp.bitwise_or(
            er[0:1, :], jnp.left_shift(er[1:2, :], 16)
        )

    return pl.pallas_call(
        body, out_shape=jax.ShapeDtypeStruct((1, E), jnp.int32)
    )(edge_index)


def _tc_first(degp, x, W1):
    F, N = W1.shape[1], x.shape[0]

    def body(dp, xr, wr, y_ref, dinv_ref):
        deg = jnp.sum(dp[...], axis=0, keepdims=True) + 1.0
        dinv = lax.rsqrt(jnp.maximum(deg, 1e-12))
        y = lax.dot_general(
            wr[...], xr[...], (((0,), (1,)), ((), ())),
            preferred_element_type=_F32,
        ) * dinv
        y_ref[...] = y
        dinv_ref[...] = dinv

    return pl.pallas_call(
        body,
        out_shape=(
            jax.ShapeDtypeStruct((F, N), _F32),
            jax.ShapeDtypeStruct((1, N), _F32),
        ),
    )(degp, x, W1)


def _tc_mid(agg, yT, dinv, WT, bcol):
    F, N = WT.shape[0], yT.shape[1]

    def body(ar, yr, dr, wr, br, o_ref):
        agg = jnp.sum(ar[...], axis=0)
        h = jnp.maximum(dr[...] * (agg + yr[...]) + br[...], 0.0)
        o_ref[...] = jnp.dot(wr[...], h, preferred_element_type=_F32) * dr[...]

    return pl.pallas_call(
        body, out_shape=jax.ShapeDtypeStruct((F, N), _F32)
    )(agg, yT, dinv, WT, bcol)


def _tc_last(agg, yT, dinv, bcol):
    F, N = yT.shape

    def body(ar, yr, dr, br, emb_ref, gm_ref):
        embT = dr[...] * (jnp.sum(ar[...], axis=0) + yr[...]) + br[...]
        emb_ref[...] = embT
        gm_ref[...] = jnp.mean(embT, axis=1, keepdims=True)

    return pl.pallas_call(
        body,
        out_shape=(
            jax.ShapeDtypeStruct((F, N), _F32),
            jax.ShapeDtypeStruct((F, 1), _F32),
        ),
    )(agg, yT, dinv, bcol)


def kernel(x, edge_index, edge_attr, W1, b1, W2, b2, W3, b3):
    N, D = x.shape
    E = edge_index.shape[1]
    H = W1.shape[1]
    O = W3.shape[1]

    dst = edge_index[1]
    ew = edge_attr[:, 0]
    W2T, W3T = W2.T, W3.T
    b1c, b2c, b3c = b1.reshape(H, 1), b2.reshape(H, 1), b3.reshape(O, 1)

    sd = _tc_pack(edge_index).reshape(E)
    degp = _make_deg(E, N)(dst, ew)
    y1T, dinv = _tc_first(degp, x, W1)

    agg_h = _make_agg(E, N, H)
    agg1 = agg_h(sd, ew, y1T)
    y2T = _tc_mid(agg1, y1T, dinv, W2T, b1c)
    agg2 = agg_h(sd, ew, y2T)
    y3T = _tc_mid(agg2, y2T, dinv, W3T, b2c)
    agg3 = _make_agg(E, N, O)(sd, ew, y3T)
    embT, gsum = _tc_last(agg3, y3T, dinv, b3c)

    return embT.T, gsum.reshape(1, O)
